# initial kernel scaffold (unmeasured)
import jax
import jax.numpy as jnp
from jax import lax
from jax.experimental import pallas as pl
from jax.experimental.pallas import tpu as pltpu

N_DEV = 4


def kernel(A, B):
    m, _ = A.shape
    _, n = B.shape
    ch = m // N_DEV

    A16 = A.astype(jnp.bfloat16)
    B16 = B.astype(jnp.bfloat16)

    def body(a_ref, b_ref, out_ref,
             rs_send, rs_recv, ag_send, ag_recv,
             rs_send_sem, rs_recv_sems, ag_send_sem, ag_recv_sems):
        my = lax.axis_index("i")
        left = lax.rem(my + N_DEV - 1, N_DEV)
        right = lax.rem(my + 1, N_DEV)

        out_ref[...] = jnp.dot(
            a_ref[...], b_ref[...], preferred_element_type=jnp.float32
        )

        barrier_sem = pltpu.get_barrier_semaphore()
        for nbr in (left, right):
            pl.semaphore_signal(
                barrier_sem, inc=1,
                device_id=(nbr,), device_id_type=pl.DeviceIdType.MESH,
            )
        pl.semaphore_wait(barrier_sem, 2)

        for s in range(N_DEV - 1):
            send_c = lax.rem(my - s + 2 * N_DEV, N_DEV)
            recv_c = lax.rem(my - s - 1 + 2 * N_DEV, N_DEV)
            rs_send[...] = out_ref[pl.ds(send_c * ch, ch), :].astype(jnp.bfloat16)
            rdma = pltpu.make_async_remote_copy(
                src_ref=rs_send,
                dst_ref=rs_recv.at[s],
                send_sem=rs_send_sem,
                recv_sem=rs_recv_sems.at[s],
                device_id=(right,),
                device_id_type=pl.DeviceIdType.MESH,
            )
            rdma.start()
            rdma.wait()
            out_ref[pl.ds(recv_c * ch, ch), :] += rs_recv[s].astype(jnp.float32)

        for t in range(N_DEV - 1):
            send_c = lax.rem(my + 1 - t + 2 * N_DEV, N_DEV)
            recv_c = lax.rem(my - t + 2 * N_DEV, N_DEV)
            if t == 0:
                ag_send[...] = out_ref[pl.ds(send_c * ch, ch), :].astype(jnp.bfloat16)
                src = ag_send
            else:
                src = ag_recv.at[t - 1]
            rdma = pltpu.make_async_remote_copy(
                src_ref=src,
                dst_ref=ag_recv.at[t],
                send_sem=ag_send_sem,
                recv_sem=ag_recv_sems.at[t],
                device_id=(right,),
                device_id_type=pl.DeviceIdType.MESH,
            )
            rdma.start()
            rdma.wait()
            out_ref[pl.ds(recv_c * ch, ch), :] = ag_recv[t].astype(jnp.float32)

    return pl.pallas_call(
        body,
        out_shape=jax.ShapeDtypeStruct((m, n), jnp.float32),
        in_specs=[
            pl.BlockSpec(memory_space=pltpu.VMEM),
            pl.BlockSpec(memory_space=pltpu.VMEM),
        ],
        out_specs=pl.BlockSpec(memory_space=pltpu.VMEM),
        scratch_shapes=[
            pltpu.VMEM((ch, n), jnp.bfloat16),
            pltpu.VMEM((N_DEV - 1, ch, n), jnp.bfloat16),
            pltpu.VMEM((ch, n), jnp.bfloat16),
            pltpu.VMEM((N_DEV - 1, ch, n), jnp.bfloat16),
            pltpu.SemaphoreType.DMA,
            pltpu.SemaphoreType.DMA((N_DEV - 1,)),
            pltpu.SemaphoreType.DMA,
            pltpu.SemaphoreType.DMA((N_DEV - 1,)),
        ],
        compiler_params=pltpu.CompilerParams(collective_id=0),
    )(A16, B16)


# baseline (device time: 191109 ns/iter reference)
import jax
import jax.numpy as jnp
from jax import lax
from jax.experimental import pallas as pl
from jax.experimental.pallas import tpu as pltpu

N_DEV = 4


def kernel(A, B):
    m, _ = A.shape
    _, n = B.shape
    ch = m // N_DEV

    A16 = A.astype(jnp.bfloat16)
    B16 = B.astype(jnp.bfloat16)

    def body(a_ref, b_ref, out_ref,
             rs_send, rs_recv, ag_send, ag_recv,
             rs_send_sem, rs_recv_sems, ag_send_sem, ag_recv_sems):
        my = lax.axis_index("i")
        left = lax.rem(my + N_DEV - 1, N_DEV)
        right = lax.rem(my + 1, N_DEV)

        out_ref[...] = jnp.dot(
            a_ref[...], b_ref[...], preferred_element_type=jnp.float32
        )

        barrier_sem = pltpu.get_barrier_semaphore()
        for nbr in (left, right):
            pl.semaphore_signal(
                barrier_sem, inc=1,
                device_id=(nbr,), device_id_type=pl.DeviceIdType.MESH,
            )
        pl.semaphore_wait(barrier_sem, 2)

        for s in range(N_DEV - 1):
            send_c = lax.rem(my - s + 2 * N_DEV, N_DEV)
            recv_c = lax.rem(my - s - 1 + 2 * N_DEV, N_DEV)
            rs_send[...] = out_ref[pl.ds(send_c * ch, ch), :].astype(jnp.bfloat16)
            rdma = pltpu.make_async_remote_copy(
                src_ref=rs_send,
                dst_ref=rs_recv.at[s],
                send_sem=rs_send_sem,
                recv_sem=rs_recv_sems.at[s],
                device_id=(right,),
                device_id_type=pl.DeviceIdType.MESH,
            )
            rdma.start()
            rdma.wait()
            out_ref[pl.ds(recv_c * ch, ch), :] += rs_recv[s].astype(jnp.float32)

        for t in range(N_DEV - 1):
            send_c = lax.rem(my + 1 - t + 2 * N_DEV, N_DEV)
            recv_c = lax.rem(my - t + 2 * N_DEV, N_DEV)
            if t == 0:
                ag_send[...] = out_ref[pl.ds(send_c * ch, ch), :].astype(jnp.bfloat16)
                src = ag_send
            else:
                src = ag_recv.at[t - 1]
            rdma = pltpu.make_async_remote_copy(
                src_ref=src,
                dst_ref=ag_recv.at[t],
                send_sem=ag_send_sem,
                recv_sem=ag_recv_sems.at[t],
                device_id=(right,),
                device_id_type=pl.DeviceIdType.MESH,
            )
            rdma.start()
            rdma.wait()
            out_ref[pl.ds(recv_c * ch, ch), :] = ag_recv[t].astype(jnp.float32)

    return pl.pallas_call(
        body,
        out_shape=jax.ShapeDtypeStruct((m, n), jnp.float32),
        in_specs=[
            pl.BlockSpec(memory_space=pltpu.VMEM),
            pl.BlockSpec(memory_space=pltpu.VMEM),
        ],
        out_specs=pl.BlockSpec(memory_space=pltpu.VMEM),
        scratch_shapes=[
            pltpu.VMEM((ch, n), jnp.bfloat16),
            pltpu.VMEM((N_DEV - 1, ch, n), jnp.bfloat16),
            pltpu.VMEM((ch, n), jnp.bfloat16),
            pltpu.VMEM((N_DEV - 1, ch, n), jnp.bfloat16),
            pltpu.SemaphoreType.DMA,
            pltpu.SemaphoreType.DMA((N_DEV - 1,)),
            pltpu.SemaphoreType.DMA,
            pltpu.SemaphoreType.DMA((N_DEV - 1,)),
        ],
        compiler_params=pltpu.CompilerParams(
            collective_id=0, vmem_limit_bytes=64 * 1024 * 1024
        ),
    )(A16, B16)


# device time: 116912 ns/iter; 1.6346x vs baseline; 1.6346x over previous
import jax
import jax.numpy as jnp
from jax import lax
from jax.experimental import pallas as pl
from jax.experimental.pallas import tpu as pltpu

N_DEV = 4


def kernel(A, B):
    m, _ = A.shape
    _, n = B.shape
    ch = m // N_DEV
    nh = n // 2

    A16 = A.astype(jnp.bfloat16)
    B16 = B.astype(jnp.bfloat16)

    def body(a_ref, b_ref, out_ref,
             rs_send_r, rs_send_l, rs_recv_r, rs_recv_l,
             ag_send_r, ag_send_l, ag_recv_r, ag_recv_l,
             rs_ss_r, rs_ss_l, rs_rs_r, rs_rs_l,
             ag_ss_r, ag_ss_l, ag_rs_r, ag_rs_l):
        my = lax.axis_index("i")
        left = lax.rem(my + N_DEV - 1, N_DEV)
        right = lax.rem(my + 1, N_DEV)

        def rows(c):
            return pl.ds(lax.rem(c + 2 * N_DEV, N_DEV) * ch, ch)

        def dot_chunk(c):
            out_ref[rows(c), :] = jnp.dot(
                a_ref[rows(c), :], b_ref[...],
                preferred_element_type=jnp.float32,
            )

        dot_chunk(my)

        barrier_sem = pltpu.get_barrier_semaphore()
        for nbr in (left, right):
            pl.semaphore_signal(
                barrier_sem, inc=1,
                device_id=(nbr,), device_id_type=pl.DeviceIdType.MESH,
            )
        pl.semaphore_wait(barrier_sem, 2)

        for s in range(N_DEV - 1):
            rs_send_r[...] = out_ref[rows(my - s), nh:].astype(jnp.bfloat16)
            rs_send_l[...] = out_ref[rows(my + s), :nh].astype(jnp.bfloat16)
            rd_r = pltpu.make_async_remote_copy(
                src_ref=rs_send_r, dst_ref=rs_recv_r.at[s],
                send_sem=rs_ss_r, recv_sem=rs_rs_r.at[s],
                device_id=(right,), device_id_type=pl.DeviceIdType.MESH,
            )
            rd_l = pltpu.make_async_remote_copy(
                src_ref=rs_send_l, dst_ref=rs_recv_l.at[s],
                send_sem=rs_ss_l, recv_sem=rs_rs_l.at[s],
                device_id=(left,), device_id_type=pl.DeviceIdType.MESH,
            )
            rd_r.start()
            rd_l.start()
            if s == 0:
                for dc in (1, 2, 3):
                    dot_chunk(my + dc)
            rd_r.wait()
            rd_l.wait()
            out_ref[rows(my - s - 1), nh:] += rs_recv_r[s].astype(jnp.float32)
            out_ref[rows(my + s + 1), :nh] += rs_recv_l[s].astype(jnp.float32)

        for t in range(N_DEV - 1):
            if t == 0:
                ag_send_r[...] = out_ref[rows(my + 1), nh:].astype(jnp.bfloat16)
                ag_send_l[...] = out_ref[rows(my - 1), :nh].astype(jnp.bfloat16)
                src_r, src_l = ag_send_r, ag_send_l
            else:
                src_r, src_l = ag_recv_r.at[t - 1], ag_recv_l.at[t - 1]
            ad_r = pltpu.make_async_remote_copy(
                src_ref=src_r, dst_ref=ag_recv_r.at[t],
                send_sem=ag_ss_r, recv_sem=ag_rs_r.at[t],
                device_id=(right,), device_id_type=pl.DeviceIdType.MESH,
            )
            ad_l = pltpu.make_async_remote_copy(
                src_ref=src_l, dst_ref=ag_recv_l.at[t],
                send_sem=ag_ss_l, recv_sem=ag_rs_l.at[t],
                device_id=(left,), device_id_type=pl.DeviceIdType.MESH,
            )
            ad_r.start()
            ad_l.start()
            ad_r.wait()
            ad_l.wait()
            out_ref[rows(my - t), nh:] = ag_recv_r[t].astype(jnp.float32)
            out_ref[rows(my + t), :nh] = ag_recv_l[t].astype(jnp.float32)

    return pl.pallas_call(
        body,
        out_shape=jax.ShapeDtypeStruct((m, n), jnp.float32),
        in_specs=[
            pl.BlockSpec(memory_space=pltpu.VMEM),
            pl.BlockSpec(memory_space=pltpu.VMEM),
        ],
        out_specs=pl.BlockSpec(memory_space=pltpu.VMEM),
        scratch_shapes=[
            pltpu.VMEM((ch, n - nh), jnp.bfloat16),
            pltpu.VMEM((ch, nh), jnp.bfloat16),
            pltpu.VMEM((N_DEV - 1, ch, n - nh), jnp.bfloat16),
            pltpu.VMEM((N_DEV - 1, ch, nh), jnp.bfloat16),
            pltpu.VMEM((ch, n - nh), jnp.bfloat16),
            pltpu.VMEM((ch, nh), jnp.bfloat16),
            pltpu.VMEM((N_DEV - 1, ch, n - nh), jnp.bfloat16),
            pltpu.VMEM((N_DEV - 1, ch, nh), jnp.bfloat16),
            pltpu.SemaphoreType.DMA,
            pltpu.SemaphoreType.DMA,
            pltpu.SemaphoreType.DMA((N_DEV - 1,)),
            pltpu.SemaphoreType.DMA((N_DEV - 1,)),
            pltpu.SemaphoreType.DMA,
            pltpu.SemaphoreType.DMA,
            pltpu.SemaphoreType.DMA((N_DEV - 1,)),
            pltpu.SemaphoreType.DMA((N_DEV - 1,)),
        ],
        compiler_params=pltpu.CompilerParams(
            collective_id=0, vmem_limit_bytes=64 * 1024 * 1024
        ),
    )(A16, B16)


# device time: 106686 ns/iter; 1.7913x vs baseline; 1.0959x over previous
import jax
import jax.numpy as jnp
from jax import lax
from jax.experimental import pallas as pl
from jax.experimental.pallas import tpu as pltpu

N_DEV = 4
N_SUB = 2


def kernel(A, B):
    m, _ = A.shape
    _, n = B.shape
    ch = m // N_DEV
    nh = n // 2
    sq = nh // N_SUB

    A16 = A.astype(jnp.bfloat16)
    B16 = B.astype(jnp.bfloat16)

    n_msg = (N_DEV - 1) * N_SUB

    def body(a_ref, b_ref, out_ref,
             rs_send_r, rs_recv_r, ag_send_r, ag_recv_r,
             rs_send_l, rs_recv_l, ag_send_l, ag_recv_l,
             rs_ss_r, rs_rs_r, ag_ss_r, ag_rs_r,
             rs_ss_l, rs_rs_l, ag_ss_l, ag_rs_l):
        my = lax.axis_index("i")
        left = lax.rem(my + N_DEV - 1, N_DEV)
        right = lax.rem(my + 1, N_DEV)

        def rows(c):
            return pl.ds(lax.rem(c + 2 * N_DEV, N_DEV) * ch, ch)

        def dot_chunk(c):
            out_ref[rows(c), :] = jnp.dot(
                a_ref[rows(c), :], b_ref[...],
                preferred_element_type=jnp.float32,
            )

        D = (
            dict(rs_send=rs_send_r, rs_recv=rs_recv_r, ag_send=ag_send_r,
                 ag_recv=ag_recv_r, rs_ss=rs_ss_r, rs_rs=rs_rs_r,
                 ag_ss=ag_ss_r, ag_rs=ag_rs_r, nbr=right, col0=nh, sgn=1),
            dict(rs_send=rs_send_l, rs_recv=rs_recv_l, ag_send=ag_send_l,
                 ag_recv=ag_recv_l, rs_ss=rs_ss_l, rs_rs=rs_rs_l,
                 ag_ss=ag_ss_l, ag_rs=ag_rs_l, nbr=left, col0=0, sgn=-1),
        )

        def cols(d, u):
            return slice(d["col0"] + u * sq, d["col0"] + (u + 1) * sq)

        def rs_desc(d, s, u):
            i = s * N_SUB + u
            return pltpu.make_async_remote_copy(
                src_ref=d["rs_send"].at[i], dst_ref=d["rs_recv"].at[i],
                send_sem=d["rs_ss"].at[i], recv_sem=d["rs_rs"].at[i],
                device_id=(d["nbr"],), device_id_type=pl.DeviceIdType.MESH,
            )

        def ag_desc(d, t, u):
            i = t * N_SUB + u
            src = d["ag_send"].at[u] if t == 0 else d["ag_recv"].at[i - N_SUB]
            return pltpu.make_async_remote_copy(
                src_ref=src, dst_ref=d["ag_recv"].at[i],
                send_sem=d["ag_ss"].at[i], recv_sem=d["ag_rs"].at[i],
                device_id=(d["nbr"],), device_id_type=pl.DeviceIdType.MESH,
            )

        def rs_stage_start(d, s, u):
            c = my - d["sgn"] * s
            d["rs_send"][s * N_SUB + u] = (
                out_ref[rows(c), cols(d, u)].astype(jnp.bfloat16)
            )
            rs_desc(d, s, u).start()

        dot_chunk(my)

        barrier_sem = pltpu.get_barrier_semaphore()
        for nbr in (left, right):
            pl.semaphore_signal(
                barrier_sem, inc=1,
                device_id=(nbr,), device_id_type=pl.DeviceIdType.MESH,
            )
        pl.semaphore_wait(barrier_sem, 2)

        for d in D:
            for u in range(N_SUB):
                rs_stage_start(d, 0, u)

        dot_chunk(my + 1)
        dot_chunk(my - 1)

        for s in range(N_DEV - 1):
            if s == 1:
                dot_chunk(my + 2)
            for u in range(N_SUB):
                for d in D:
                    rs_desc(d, s, u).wait_recv()
                    rc = my - d["sgn"] * (s + 1)
                    out_ref[rows(rc), cols(d, u)] += (
                        d["rs_recv"][s * N_SUB + u].astype(jnp.float32)
                    )
                    if s < N_DEV - 2:
                        rs_stage_start(d, s + 1, u)
                    else:
                        d["ag_send"][u] = (
                            out_ref[rows(my + d["sgn"]), cols(d, u)]
                            .astype(jnp.bfloat16)
                        )
                        ag_desc(d, 0, u).start()

        for t in range(N_DEV - 1):
            for u in range(N_SUB):
                for d in D:
                    ag_desc(d, t, u).wait_recv()
                    if t < N_DEV - 2:
                        ag_desc(d, t + 1, u).start()
                    rc = my - d["sgn"] * t
                    out_ref[rows(rc), cols(d, u)] = (
                        d["ag_recv"][t * N_SUB + u].astype(jnp.float32)
                    )

        for d in D:
            for s in range(N_DEV - 1):
                for u in range(N_SUB):
                    rs_desc(d, s, u).wait_send()
                    ag_desc(d, s, u).wait_send()

    return pl.pallas_call(
        body,
        out_shape=jax.ShapeDtypeStruct((m, n), jnp.float32),
        in_specs=[
            pl.BlockSpec(memory_space=pltpu.VMEM),
            pl.BlockSpec(memory_space=pltpu.VMEM),
        ],
        out_specs=pl.BlockSpec(memory_space=pltpu.VMEM),
        scratch_shapes=[
            pltpu.VMEM((n_msg, ch, sq), jnp.bfloat16),
            pltpu.VMEM((n_msg, ch, sq), jnp.bfloat16),
            pltpu.VMEM((N_SUB, ch, sq), jnp.bfloat16),
            pltpu.VMEM((n_msg, ch, sq), jnp.bfloat16),
            pltpu.VMEM((n_msg, ch, sq), jnp.bfloat16),
            pltpu.VMEM((n_msg, ch, sq), jnp.bfloat16),
            pltpu.VMEM((N_SUB, ch, sq), jnp.bfloat16),
            pltpu.VMEM((n_msg, ch, sq), jnp.bfloat16),
            pltpu.SemaphoreType.DMA((n_msg,)),
            pltpu.SemaphoreType.DMA((n_msg,)),
            pltpu.SemaphoreType.DMA((n_msg,)),
            pltpu.SemaphoreType.DMA((n_msg,)),
            pltpu.SemaphoreType.DMA((n_msg,)),
            pltpu.SemaphoreType.DMA((n_msg,)),
            pltpu.SemaphoreType.DMA((n_msg,)),
            pltpu.SemaphoreType.DMA((n_msg,)),
        ],
        compiler_params=pltpu.CompilerParams(
            collective_id=0, vmem_limit_bytes=64 * 1024 * 1024
        ),
    )(A16, B16)


# device time: 99577 ns/iter; 1.9192x vs baseline; 1.0714x over previous
import jax
import jax.numpy as jnp
from jax import lax
from jax.experimental import pallas as pl
from jax.experimental.pallas import tpu as pltpu

N_DEV = 4
N_SUB = 2


def kernel(A, B):
    m, _ = A.shape
    _, n = B.shape
    ch = m // N_DEV
    nh = n // 2
    sq = nh // N_SUB

    n_msg = (N_DEV - 1) * N_SUB

    def body(a_ref, b_ref, out_ref, b16,
             rs_send_r, rs_recv_r, ag_send_r, ag_recv_r,
             rs_send_l, rs_recv_l, ag_send_l, ag_recv_l,
             rs_ss_r, rs_rs_r, ag_ss_r, ag_rs_r,
             rs_ss_l, rs_rs_l, ag_ss_l, ag_rs_l):
        my = lax.axis_index("i")
        left = lax.rem(my + N_DEV - 1, N_DEV)
        right = lax.rem(my + 1, N_DEV)

        def rows(c):
            return pl.ds(lax.rem(c + 2 * N_DEV, N_DEV) * ch, ch)

        def dot_chunk(c):
            out_ref[rows(c), :] = jnp.dot(
                a_ref[rows(c), :].astype(jnp.bfloat16), b16[...],
                preferred_element_type=jnp.float32,
            )

        D = (
            dict(rs_send=rs_send_r, rs_recv=rs_recv_r, ag_send=ag_send_r,
                 ag_recv=ag_recv_r, rs_ss=rs_ss_r, rs_rs=rs_rs_r,
                 ag_ss=ag_ss_r, ag_rs=ag_rs_r, nbr=right, col0=nh, sgn=1),
            dict(rs_send=rs_send_l, rs_recv=rs_recv_l, ag_send=ag_send_l,
                 ag_recv=ag_recv_l, rs_ss=rs_ss_l, rs_rs=rs_rs_l,
                 ag_ss=ag_ss_l, ag_rs=ag_rs_l, nbr=left, col0=0, sgn=-1),
        )

        def cols(d, u):
            return slice(d["col0"] + u * sq, d["col0"] + (u + 1) * sq)

        def rs_desc(d, s, u):
            i = s * N_SUB + u
            return pltpu.make_async_remote_copy(
                src_ref=d["rs_send"].at[i], dst_ref=d["rs_recv"].at[i],
                send_sem=d["rs_ss"].at[i], recv_sem=d["rs_rs"].at[i],
                device_id=(d["nbr"],), device_id_type=pl.DeviceIdType.MESH,
            )

        def ag_desc(d, t, u):
            i = t * N_SUB + u
            src = d["ag_send"].at[u] if t == 0 else d["ag_recv"].at[i - N_SUB]
            return pltpu.make_async_remote_copy(
                src_ref=src, dst_ref=d["ag_recv"].at[i],
                send_sem=d["ag_ss"].at[i], recv_sem=d["ag_rs"].at[i],
                device_id=(d["nbr"],), device_id_type=pl.DeviceIdType.MESH,
            )

        def rs_stage_start(d, s, u):
            c = my - d["sgn"] * s
            d["rs_send"][s * N_SUB + u] = (
                out_ref[rows(c), cols(d, u)].astype(jnp.bfloat16)
            )
            rs_desc(d, s, u).start()

        b16[...] = b_ref[...].astype(jnp.bfloat16)

        dot_chunk(my)

        barrier_sem = pltpu.get_barrier_semaphore()
        for nbr in (left, right):
            pl.semaphore_signal(
                barrier_sem, inc=1,
                device_id=(nbr,), device_id_type=pl.DeviceIdType.MESH,
            )
        pl.semaphore_wait(barrier_sem, 2)

        for d in D:
            for u in range(N_SUB):
                rs_stage_start(d, 0, u)

        dot_chunk(my + 1)
        dot_chunk(my - 1)

        for s in range(N_DEV - 1):
            if s == 1:
                dot_chunk(my + 2)
            for u in range(N_SUB):
                for d in D:
                    rs_desc(d, s, u).wait_recv()
                    rc = my - d["sgn"] * (s + 1)
                    out_ref[rows(rc), cols(d, u)] += (
                        d["rs_recv"][s * N_SUB + u].astype(jnp.float32)
                    )
                    if s < N_DEV - 2:
                        rs_stage_start(d, s + 1, u)
                    else:
                        d["ag_send"][u] = (
                            out_ref[rows(my + d["sgn"]), cols(d, u)]
                            .astype(jnp.bfloat16)
                        )
                        ag_desc(d, 0, u).start()

        for t in range(N_DEV - 1):
            for u in range(N_SUB):
                for d in D:
                    ag_desc(d, t, u).wait_recv()
                    if t < N_DEV - 2:
                        ag_desc(d, t + 1, u).start()
                    rc = my - d["sgn"] * t
                    out_ref[rows(rc), cols(d, u)] = (
                        d["ag_recv"][t * N_SUB + u].astype(jnp.float32)
                    )

        for d in D:
            for s in range(N_DEV - 1):
                for u in range(N_SUB):
                    rs_desc(d, s, u).wait_send()
                    ag_desc(d, s, u).wait_send()

    return pl.pallas_call(
        body,
        out_shape=jax.ShapeDtypeStruct((m, n), jnp.float32),
        in_specs=[
            pl.BlockSpec(memory_space=pltpu.VMEM),
            pl.BlockSpec(memory_space=pltpu.VMEM),
        ],
        out_specs=pl.BlockSpec(memory_space=pltpu.VMEM),
        scratch_shapes=[
            pltpu.VMEM(B.shape, jnp.bfloat16),
            pltpu.VMEM((n_msg, ch, sq), jnp.bfloat16),
            pltpu.VMEM((n_msg, ch, sq), jnp.bfloat16),
            pltpu.VMEM((N_SUB, ch, sq), jnp.bfloat16),
            pltpu.VMEM((n_msg, ch, sq), jnp.bfloat16),
            pltpu.VMEM((n_msg, ch, sq), jnp.bfloat16),
            pltpu.VMEM((n_msg, ch, sq), jnp.bfloat16),
            pltpu.VMEM((N_SUB, ch, sq), jnp.bfloat16),
            pltpu.VMEM((n_msg, ch, sq), jnp.bfloat16),
            pltpu.SemaphoreType.DMA((n_msg,)),
            pltpu.SemaphoreType.DMA((n_msg,)),
            pltpu.SemaphoreType.DMA((n_msg,)),
            pltpu.SemaphoreType.DMA((n_msg,)),
            pltpu.SemaphoreType.DMA((n_msg,)),
            pltpu.SemaphoreType.DMA((n_msg,)),
            pltpu.SemaphoreType.DMA((n_msg,)),
            pltpu.SemaphoreType.DMA((n_msg,)),
        ],
        compiler_params=pltpu.CompilerParams(
            collective_id=0, vmem_limit_bytes=96 * 1024 * 1024
        ),
    )(A, B)
